# Initial kernel scaffold; baseline (speedup 1.0000x reference)
import jax
import jax.numpy as jnp
from jax.experimental import pallas as pl


def kernel(x, W):
    raise NotImplementedError("write your pallas kernel here")


# speedup vs baseline: 5.4325x; 5.4325x over previous
import functools

import jax
import jax.numpy as jnp
from jax.experimental import pallas as pl

_NUM_EXPERTS = 8
_EXPERTS_PER_GROUP = 4
_ROUTE_SCALE = 2.5
_NEG_INF = float("-inf")


def _gate_kernel(x_ref, w_ref, wout_ref, iout_ref):
    s = jnp.dot(x_ref[...], w_ref[...], preferred_element_type=jnp.float32)
    s = jax.nn.sigmoid(s)

    lane = jax.lax.broadcasted_iota(jnp.int32, s.shape, 1)
    in_g1 = lane >= _EXPERTS_PER_GROUP
    gmax0 = jnp.max(jnp.where(in_g1, _NEG_INF, s), axis=1, keepdims=True)
    gmax1 = jnp.max(jnp.where(in_g1, s, _NEG_INF), axis=1, keepdims=True)
    sel_g1 = gmax1 > gmax0
    masked = jnp.where(in_g1 == sel_g1, s, _NEG_INF)

    m1 = jnp.max(masked, axis=1, keepdims=True)
    i1 = jnp.min(jnp.where(masked == m1, lane, _NUM_EXPERTS), axis=1, keepdims=True)
    masked2 = jnp.where(lane == i1, _NEG_INF, masked)
    m2 = jnp.max(masked2, axis=1, keepdims=True)
    i2 = jnp.min(jnp.where(masked2 == m2, lane, _NUM_EXPERTS), axis=1, keepdims=True)

    inv = _ROUTE_SCALE / (m1 + m2)
    wout_ref[...] = jnp.concatenate([m1 * inv, m2 * inv], axis=1)
    iout_ref[...] = jnp.concatenate([i1, i2], axis=1)


@functools.partial(jax.jit, static_argnames=())
def kernel(x, W):
    num_tokens, hidden = x.shape
    block_t = 2048
    grid = (num_tokens // block_t,)
    weights, idx = pl.pallas_call(
        _gate_kernel,
        grid=grid,
        in_specs=[
            pl.BlockSpec((block_t, hidden), lambda i: (i, 0)),
            pl.BlockSpec((hidden, _NUM_EXPERTS), lambda i: (0, 0)),
        ],
        out_specs=[
            pl.BlockSpec((block_t, 2), lambda i: (i, 0)),
            pl.BlockSpec((block_t, 2), lambda i: (i, 0)),
        ],
        out_shape=[
            jax.ShapeDtypeStruct((num_tokens, 2), jnp.float32),
            jax.ShapeDtypeStruct((num_tokens, 2), jnp.int32),
        ],
    )(x, W)
    return weights.astype(x.dtype), idx


# speedup vs baseline: 5.6763x; 1.0449x over previous
import functools

import jax
import jax.numpy as jnp
from jax.experimental import pallas as pl

_NUM_EXPERTS = 8
_EXPERTS_PER_GROUP = 4
_ROUTE_SCALE = 2.5
_NEG_INF = float("-inf")


def _gate_kernel(x_ref, w_ref, wout_ref, iout_ref):
    s = jnp.dot(x_ref[...], w_ref[...], preferred_element_type=jnp.float32)
    s = jax.nn.sigmoid(s)

    lane = jax.lax.broadcasted_iota(jnp.int32, s.shape, 1)
    in_g1 = lane >= _EXPERTS_PER_GROUP
    gmax0 = jnp.max(jnp.where(in_g1, _NEG_INF, s), axis=1, keepdims=True)
    gmax1 = jnp.max(jnp.where(in_g1, s, _NEG_INF), axis=1, keepdims=True)
    sel_g1 = gmax1 > gmax0
    masked = jnp.where(in_g1 == sel_g1, s, _NEG_INF)

    m1 = jnp.max(masked, axis=1, keepdims=True)
    i1 = jnp.min(jnp.where(masked == m1, lane, _NUM_EXPERTS), axis=1, keepdims=True)
    masked2 = jnp.where(lane == i1, _NEG_INF, masked)
    m2 = jnp.max(masked2, axis=1, keepdims=True)
    i2 = jnp.min(jnp.where(masked2 == m2, lane, _NUM_EXPERTS), axis=1, keepdims=True)

    inv = _ROUTE_SCALE / (m1 + m2)
    wout_ref[...] = jnp.concatenate([m1 * inv, m2 * inv], axis=1)
    iout_ref[...] = jnp.concatenate([i1, i2], axis=1)


@functools.partial(jax.jit, static_argnames=())
def kernel(x, W):
    num_tokens, hidden = x.shape
    block_t = 4096
    grid = (num_tokens // block_t,)
    weights, idx = pl.pallas_call(
        _gate_kernel,
        grid=grid,
        in_specs=[
            pl.BlockSpec((block_t, hidden), lambda i: (i, 0)),
            pl.BlockSpec((hidden, _NUM_EXPERTS), lambda i: (0, 0)),
        ],
        out_specs=[
            pl.BlockSpec((block_t, 2), lambda i: (i, 0)),
            pl.BlockSpec((block_t, 2), lambda i: (i, 0)),
        ],
        out_shape=[
            jax.ShapeDtypeStruct((num_tokens, 2), jnp.float32),
            jax.ShapeDtypeStruct((num_tokens, 2), jnp.int32),
        ],
    )(x, W)
    return weights.astype(x.dtype), idx


# speedup vs baseline: 11.6002x; 2.0436x over previous
import functools

import jax
import jax.numpy as jnp
from jax.experimental import pallas as pl

_NUM_EXPERTS = 8
_EXPERTS_PER_GROUP = 4
_ROUTE_SCALE = 2.5
_NEG_INF = float("-inf")


def _gate_kernel(x_ref, w_ref, wout_ref, iout_ref):
    st = jax.lax.dot_general(
        w_ref[...], x_ref[...],
        dimension_numbers=(((0,), (1,)), ((), ())),
        preferred_element_type=jnp.float32,
    )
    st = jax.nn.sigmoid(st)

    sub = jax.lax.broadcasted_iota(jnp.int32, st.shape, 0)
    in_g1 = sub >= _EXPERTS_PER_GROUP
    gmax0 = jnp.max(jnp.where(in_g1, _NEG_INF, st), axis=0, keepdims=True)
    gmax1 = jnp.max(jnp.where(in_g1, st, _NEG_INF), axis=0, keepdims=True)
    sel_g1 = gmax1 > gmax0
    masked = jnp.where(in_g1 == sel_g1, st, _NEG_INF)

    m1 = jnp.max(masked, axis=0, keepdims=True)
    i1 = jnp.min(jnp.where(masked == m1, sub, _NUM_EXPERTS), axis=0, keepdims=True)
    masked2 = jnp.where(sub == i1, _NEG_INF, masked)
    m2 = jnp.max(masked2, axis=0, keepdims=True)
    i2 = jnp.min(jnp.where(masked2 == m2, sub, _NUM_EXPERTS), axis=0, keepdims=True)

    inv = _ROUTE_SCALE / (m1 + m2)
    wout_ref[...] = jnp.concatenate([m1 * inv, m2 * inv], axis=0)
    iout_ref[...] = jnp.concatenate([i1, i2], axis=0)


@functools.partial(jax.jit, static_argnames=())
def kernel(x, W):
    num_tokens, hidden = x.shape
    block_t = 4096
    grid = (num_tokens // block_t,)
    weights_t, idx_t = pl.pallas_call(
        _gate_kernel,
        grid=grid,
        in_specs=[
            pl.BlockSpec((block_t, hidden), lambda i: (i, 0)),
            pl.BlockSpec((hidden, _NUM_EXPERTS), lambda i: (0, 0)),
        ],
        out_specs=[
            pl.BlockSpec((2, block_t), lambda i: (0, i)),
            pl.BlockSpec((2, block_t), lambda i: (0, i)),
        ],
        out_shape=[
            jax.ShapeDtypeStruct((2, num_tokens), jnp.float32),
            jax.ShapeDtypeStruct((2, num_tokens), jnp.int32),
        ],
    )(x, W)
    return weights_t.T.astype(x.dtype), idx_t.T
